# baseline (speedup 1.0000x reference)
import functools

import jax
import jax.numpy as jnp
from jax import lax
from jax.experimental import pallas as pl
from jax.experimental.pallas import tpu as pltpu
from jax.experimental.pallas import tpu_sc as plsc

N = 10000
E = 160000
D = 256
H = 128
G = 8

NC = 2
NS = 16
L = 16

K = 56
NCH = 184
EC = NCH * K
EP = NS * EC
NP = 10240
ZR = NP // NS

BR = 2000
NB = N // BR


def _edge_body(a_lo, a_hi, b_lo, b_hi, idx_all, out_lo, out_hi,
               idxb, abuf, bbuf, mbuf, agg_sh, sem_g0, sem_g1, sem_i0,
               sem_i1, sem_s0, sem_s1):
  cid = lax.axis_index("c")
  sid = lax.axis_index("s")
  sems_g = (sem_g0, sem_g1)
  sems_i = (sem_i0, sem_i1)
  sems_s = (sem_s0, sem_s1)
  GG = NCH // 4

  def run(atab, btab, out):
    @plsc.parallel_loop(0, K, unroll=4)
    def _(i):
      for q in range(H // L):
        mbuf[0, i, pl.ds(q * L, L)] = jnp.zeros((L,), jnp.float32)

    zbase = sid * ZR
    for r in range(ZR // K):
      pltpu.sync_copy(mbuf.at[0], agg_sh.at[pl.ds(zbase + r * K, K)])
    rem = ZR - (ZR // K) * K
    pltpu.sync_copy(mbuf.at[0, pl.ds(0, rem)],
                    agg_sh.at[pl.ds(zbase + (ZR // K) * K, rem)])

    ibase = sid * NCH
    pltpu.sync_copy(idx_all.at[pl.ds(ibase, 2)], idxb.at[0])
    pltpu.async_copy(atab.at[idxb.at[0, 0, 0]], abuf.at[0], sems_g[0])
    pltpu.async_copy(btab.at[idxb.at[0, 0, 1]], bbuf.at[0], sems_g[0])

    plsc.subcore_barrier()

    def quad(gg, _):
      for ib in range(2):
        g = gg * 2 + ib

        def idx_prefetch():
          pltpu.async_copy(idx_all.at[pl.ds(ibase + (g + 1) * 2, 2)],
                           idxb.at[1 - ib], sems_i[1 - ib])
        if ib == 0:
          idx_prefetch()
        else:
          pl.when(gg < GG - 1)(idx_prefetch)

        for b in range(2):
          if b == 0:
            pltpu.async_copy(atab.at[idxb.at[ib, 1, 0]], abuf.at[1],
                             sems_g[1])
            pltpu.async_copy(btab.at[idxb.at[ib, 1, 1]], bbuf.at[1],
                             sems_g[1])
          else:
            def gather_prefetch():
              pltpu.make_async_copy(
                  idx_all.at[pl.ds(ibase + (g + 1) * 2, 2)],
                  idxb.at[1 - ib], sems_i[1 - ib]).wait()
              pltpu.async_copy(atab.at[idxb.at[1 - ib, 0, 0]], abuf.at[0],
                               sems_g[0])
              pltpu.async_copy(btab.at[idxb.at[1 - ib, 0, 1]], bbuf.at[0],
                               sems_g[0])
            if ib == 0:
              gather_prefetch()
            else:
              pl.when(gg < GG - 1)(gather_prefetch)

          pltpu.make_async_copy(atab.at[idxb.at[ib, b, 0]], abuf.at[b],
                                sems_g[b]).wait()
          pltpu.make_async_copy(btab.at[idxb.at[ib, b, 1]], bbuf.at[b],
                                sems_g[b]).wait()

          def wait_prev_scatter():
            pltpu.make_async_copy(mbuf.at[b], agg_sh.at[idxb.at[ib, b, 2]],
                                  sems_s[b]).wait()
          if ib == 0:
            pl.when(gg > 0)(wait_prev_scatter)
          else:
            wait_prev_scatter()

          @plsc.parallel_loop(0, K, unroll=2)
          def _(i):
            for q in range(H // L):
              sl = pl.ds(q * L, L)
              mbuf[b, i, sl] = jnp.maximum(abuf[b, i, sl] + bbuf[b, i, sl],
                                           0.0)

          pltpu.async_copy(mbuf.at[b], agg_sh.at[idxb.at[ib, b, 2]],
                           sems_s[b], add=True)
      return 0
    lax.fori_loop(0, GG, quad, 0)

    for b in range(2):
      pltpu.make_async_copy(mbuf.at[b], agg_sh.at[idxb.at[1, b, 2]],
                            sems_s[b]).wait()

    plsc.subcore_barrier()
    pltpu.sync_copy(agg_sh.at[pl.ds(zbase, ZR)], out.at[pl.ds(zbase, ZR)])

  @pl.when(cid == 0)
  def _():
    run(a_lo, b_lo, out_lo)

  @pl.when(cid == 1)
  def _():
    run(a_hi, b_hi, out_hi)


_edge_call = functools.partial(
    pl.kernel,
    out_type=[jax.ShapeDtypeStruct((NP, H), jnp.float32),
              jax.ShapeDtypeStruct((NP, H), jnp.float32)],
    mesh=plsc.VectorSubcoreMesh(core_axis_name="c", subcore_axis_name="s"),
    scratch_types=[
        pltpu.VMEM((2, 2, 3, K), jnp.int32),
        pltpu.VMEM((2, K, H), jnp.float32),
        pltpu.VMEM((2, K, H), jnp.float32),
        pltpu.VMEM((2, K, H), jnp.float32),
        pltpu.VMEM_SHARED((NP, H), jnp.float32),
        pltpu.SemaphoreType.DMA,
        pltpu.SemaphoreType.DMA,
        pltpu.SemaphoreType.DMA,
        pltpu.SemaphoreType.DMA,
        pltpu.SemaphoreType.DMA,
        pltpu.SemaphoreType.DMA,
    ],
)(_edge_body)


def _dot(a, b):
  return jnp.dot(a, b, preferred_element_type=jnp.float32)


def _prep_body(state_ref, action_ref, wet_ref, web_ref, be_ref,
               h_ref, alo_ref, ahi_ref, blo_ref, bhi_ref):
  h = state_ref[...] + action_ref[...]
  h_ref[...] = h
  a = _dot(h, wet_ref[...])
  b = _dot(h, web_ref[...]) + be_ref[...]
  alo_ref[...] = a[:, :H]
  ahi_ref[...] = a[:, H:]
  blo_ref[...] = b[:, :H]
  bhi_ref[...] = b[:, H:]


def _update_body(h_ref, aglo_ref, aghi_ref, wnt_ref, wnblo_ref, wnbhi_ref,
                 bn_ref, wet_ref, web_ref, be_ref,
                 h1_ref, alo_ref, ahi_ref, blo_ref, bhi_ref):
  h = h_ref[...]
  u = (_dot(h, wnt_ref[...]) + _dot(aglo_ref[...], wnblo_ref[...])
       + _dot(aghi_ref[...], wnbhi_ref[...]) + bn_ref[...])
  h1 = h + jnp.maximum(u, 0.0)
  h1_ref[...] = h1
  a = _dot(h1, wet_ref[...])
  b = _dot(h1, web_ref[...]) + be_ref[...]
  alo_ref[...] = a[:, :H]
  ahi_ref[...] = a[:, H:]
  blo_ref[...] = b[:, :H]
  bhi_ref[...] = b[:, H:]


def _final_body(h_ref, aglo_ref, aghi_ref, wnt_ref, wnblo_ref, wnbhi_ref,
                bn_ref, seg_ref, w1_ref, b1_ref, w2_ref, b2_ref,
                w3_ref, b3_ref, out_ref, gsum_ref, cnt_ref):
  h = h_ref[...]
  u = (_dot(h, wnt_ref[...]) + _dot(aglo_ref[...], wnblo_ref[...])
       + _dot(aghi_ref[...], wnbhi_ref[...]) + bn_ref[...])
  h2 = h + jnp.maximum(u, 0.0)
  seg = seg_ref[0, 0, :]
  gids = lax.broadcasted_iota(jnp.int32, (G, BR), 0)
  m = (seg[None, :] == gids).astype(jnp.float32)

  @pl.when(pl.program_id(0) == 0)
  def _():
    gsum_ref[...] = jnp.zeros_like(gsum_ref)
    cnt_ref[...] = jnp.zeros_like(cnt_ref)

  gsum_ref[...] += _dot(m, h2)
  cnt_ref[...] += jnp.broadcast_to(jnp.sum(m, axis=1, keepdims=True), (G, D))

  @pl.when(pl.program_id(0) == NB - 1)
  def _():
    g = gsum_ref[...] / jnp.maximum(cnt_ref[...], 1.0)
    x = jnp.maximum(_dot(g, w1_ref[...]) + b1_ref[...], 0.0)
    x = jnp.maximum(_dot(x, w2_ref[...]) + b2_ref[...], 0.0)
    out_ref[...] = _dot(x, w3_ref[...]) + b3_ref[...]


def _row_spec(width):
  return pl.BlockSpec((BR, width), lambda i: (i, 0))


def _full_spec(shape):
  return pl.BlockSpec(shape, lambda i: tuple(0 for _ in shape))


_prep_call = pl.pallas_call(
    _prep_body,
    grid=(NB,),
    in_specs=[_row_spec(D), _row_spec(D), _full_spec((D, D)),
              _full_spec((D, D)), _full_spec((1, D))],
    out_specs=[_row_spec(D), _row_spec(H), _row_spec(H),
               _row_spec(H), _row_spec(H)],
    out_shape=[jax.ShapeDtypeStruct((N, D), jnp.float32)]
    + [jax.ShapeDtypeStruct((N, H), jnp.float32)] * 4,
)

_update_call = pl.pallas_call(
    _update_body,
    grid=(NB,),
    in_specs=[_row_spec(D), _row_spec(H), _row_spec(H),
              _full_spec((D, D)), _full_spec((H, D)), _full_spec((H, D)),
              _full_spec((1, D)), _full_spec((D, D)), _full_spec((D, D)),
              _full_spec((1, D))],
    out_specs=[_row_spec(D), _row_spec(H), _row_spec(H),
               _row_spec(H), _row_spec(H)],
    out_shape=[jax.ShapeDtypeStruct((N, D), jnp.float32)]
    + [jax.ShapeDtypeStruct((N, H), jnp.float32)] * 4,
)

_final_call = pl.pallas_call(
    _final_body,
    grid=(NB,),
    in_specs=[_row_spec(D), _row_spec(H), _row_spec(H),
              _full_spec((D, D)), _full_spec((H, D)), _full_spec((H, D)),
              _full_spec((1, D)), pl.BlockSpec((1, 1, BR), lambda i: (i, 0, 0)),
              _full_spec((D, D)), _full_spec((1, D)), _full_spec((D, D)),
              _full_spec((1, D)), _full_spec((D, H)), _full_spec((1, H))],
    out_specs=_full_spec((G, H)),
    out_shape=jax.ShapeDtypeStruct((G, H), jnp.float32),
    scratch_shapes=[pltpu.VMEM((G, D), jnp.float32),
                    pltpu.VMEM((G, D), jnp.float32)],
)


@jax.jit
def _run(state_batch, action_batch, edge_index, segment_ids,
         We0, be0, Wn0, bn0, We1, be1, Wn1, bn1, W1, b1, W2, b2, W3, b3):
  src = edge_index[0]
  dst = edge_index[1]
  zpad = jnp.zeros((EP - E,), jnp.int32)
  src_g = jnp.concatenate([src, zpad]).reshape(NS * NCH, K)
  dst_g = jnp.concatenate([dst, zpad]).reshape(NS * NCH, K)
  dst_s = jnp.concatenate([dst, jnp.full((EP - E,), N, jnp.int32)]
                          ).reshape(NS * NCH, K)
  idx_all = jnp.stack([src_g, dst_g, dst_s], axis=1)
  seg3 = segment_ids.reshape(NB, 1, BR)

  h0, a0lo, a0hi, b0lo, b0hi = _prep_call(
      state_batch, action_batch, We0[:D], We0[D:], be0.reshape(1, D))
  ag0lo, ag0hi = _edge_call(a0lo, a0hi, b0lo, b0hi, idx_all)
  h1, a1lo, a1hi, b1lo, b1hi = _update_call(
      h0, ag0lo, ag0hi, Wn0[:D], Wn0[D:D + H], Wn0[D + H:], bn0.reshape(1, D),
      We1[:D], We1[D:], be1.reshape(1, D))
  ag1lo, ag1hi = _edge_call(a1lo, a1hi, b1lo, b1hi, idx_all)
  w3p = jnp.pad(W3, ((0, 0), (0, H - 1)))
  b3p = jnp.pad(b3.reshape(1, 1), ((0, 0), (0, H - 1)))
  out = _final_call(
      h1, ag1lo, ag1hi, Wn1[:D], Wn1[D:D + H], Wn1[D + H:], bn1.reshape(1, D),
      seg3, W1, b1.reshape(1, D), W2, b2.reshape(1, D), w3p, b3p)
  return out[:, :1]


def kernel(state_batch, action_batch, edge_index, segment_ids,
           We0, be0, Wn0, bn0, We1, be1, Wn1, bn1, W1, b1, W2, b2, W3, b3):
  return _run(state_batch, action_batch, edge_index, segment_ids,
              We0, be0, Wn0, bn0, We1, be1, Wn1, bn1, W1, b1, W2, b2, W3, b3)


# speedup vs baseline: 1.1748x; 1.1748x over previous
import functools

import jax
import jax.numpy as jnp
from jax import lax
from jax.experimental import pallas as pl
from jax.experimental.pallas import tpu as pltpu
from jax.experimental.pallas import tpu_sc as plsc

N = 10000
E = 160000
D = 256
H = 128
G = 8

NC = 2
NS = 16
L = 16

K = 64
NCH = 160
EC = NCH * K
EP = NS * EC
NP = 10240
ZR = NP // NS

BR = 2000
NB = N // BR


def _edge_body(a_lo, a_hi, b_lo, b_hi, idx_all, out_lo, out_hi,
               idxb, abuf, bbuf, mbuf, agg_sh, sem_g0, sem_g1, sem_i0,
               sem_i1):
  cid = lax.axis_index("c")
  sid = lax.axis_index("s")
  sems_g = (sem_g0, sem_g1)
  sems_i = (sem_i0, sem_i1)
  GG = NCH // 4

  def run(atab, btab, out):
    @plsc.parallel_loop(0, K, unroll=4)
    def _(i):
      for q in range(H // L):
        mbuf[i, pl.ds(q * L, L)] = jnp.zeros((L,), jnp.float32)

    zbase = sid * ZR
    for r in range(ZR // K):
      pltpu.sync_copy(mbuf, agg_sh.at[pl.ds(zbase + r * K, K)])

    ibase = sid * NCH
    pltpu.sync_copy(idx_all.at[pl.ds(ibase, 2)], idxb.at[0])
    pltpu.async_copy(atab.at[idxb.at[0, 0, 0]], abuf.at[0], sems_g[0])
    pltpu.async_copy(btab.at[idxb.at[0, 0, 1]], bbuf.at[0], sems_g[0])

    plsc.subcore_barrier()

    def quad(gg, _):
      for ib in range(2):
        g = gg * 2 + ib

        def idx_prefetch():
          pltpu.async_copy(idx_all.at[pl.ds(ibase + (g + 1) * 2, 2)],
                           idxb.at[1 - ib], sems_i[1 - ib])
        if ib == 0:
          idx_prefetch()
        else:
          pl.when(gg < GG - 1)(idx_prefetch)

        for b in range(2):
          if b == 0:
            pltpu.async_copy(atab.at[idxb.at[ib, 1, 0]], abuf.at[1],
                             sems_g[1])
            pltpu.async_copy(btab.at[idxb.at[ib, 1, 1]], bbuf.at[1],
                             sems_g[1])
          else:
            def gather_prefetch():
              pltpu.make_async_copy(
                  idx_all.at[pl.ds(ibase + (g + 1) * 2, 2)],
                  idxb.at[1 - ib], sems_i[1 - ib]).wait()
              pltpu.async_copy(atab.at[idxb.at[1 - ib, 0, 0]], abuf.at[0],
                               sems_g[0])
              pltpu.async_copy(btab.at[idxb.at[1 - ib, 0, 1]], bbuf.at[0],
                               sems_g[0])
            if ib == 0:
              gather_prefetch()
            else:
              pl.when(gg < GG - 1)(gather_prefetch)

          pltpu.make_async_copy(atab.at[idxb.at[ib, b, 0]], abuf.at[b],
                                sems_g[b]).wait()
          pltpu.make_async_copy(btab.at[idxb.at[ib, b, 1]], bbuf.at[b],
                                sems_g[b]).wait()

          @plsc.parallel_loop(0, K, unroll=2)
          def _(i):
            for q in range(H // L):
              sl = pl.ds(q * L, L)
              mbuf[i, sl] = jnp.maximum(abuf[b, i, sl] + bbuf[b, i, sl],
                                        0.0)

          pltpu.sync_copy(mbuf, agg_sh.at[idxb.at[ib, b, 2]], add=True)
      return 0
    lax.fori_loop(0, GG, quad, 0)

    plsc.subcore_barrier()
    pltpu.sync_copy(agg_sh.at[pl.ds(zbase, ZR)], out.at[pl.ds(zbase, ZR)])

  @pl.when(cid == 0)
  def _():
    run(a_lo, b_lo, out_lo)

  @pl.when(cid == 1)
  def _():
    run(a_hi, b_hi, out_hi)


_edge_call = functools.partial(
    pl.kernel,
    out_type=[jax.ShapeDtypeStruct((NP, H), jnp.float32),
              jax.ShapeDtypeStruct((NP, H), jnp.float32)],
    mesh=plsc.VectorSubcoreMesh(core_axis_name="c", subcore_axis_name="s"),
    scratch_types=[
        pltpu.VMEM((2, 2, 3, K), jnp.int32),
        pltpu.VMEM((2, K, H), jnp.float32),
        pltpu.VMEM((2, K, H), jnp.float32),
        pltpu.VMEM((K, H), jnp.float32),
        pltpu.VMEM_SHARED((NP, H), jnp.float32),
        pltpu.SemaphoreType.DMA,
        pltpu.SemaphoreType.DMA,
        pltpu.SemaphoreType.DMA,
        pltpu.SemaphoreType.DMA,
    ],
)(_edge_body)


def _dot(a, b):
  return jnp.dot(a, b, preferred_element_type=jnp.float32)


def _prep_body(state_ref, action_ref, wet_ref, web_ref, be_ref,
               h_ref, alo_ref, ahi_ref, blo_ref, bhi_ref):
  h = state_ref[...] + action_ref[...]
  h_ref[...] = h
  a = _dot(h, wet_ref[...])
  b = _dot(h, web_ref[...]) + be_ref[...]
  alo_ref[...] = a[:, :H]
  ahi_ref[...] = a[:, H:]
  blo_ref[...] = b[:, :H]
  bhi_ref[...] = b[:, H:]


def _update_body(h_ref, aglo_ref, aghi_ref, wnt_ref, wnblo_ref, wnbhi_ref,
                 bn_ref, wet_ref, web_ref, be_ref,
                 h1_ref, alo_ref, ahi_ref, blo_ref, bhi_ref):
  h = h_ref[...]
  u = (_dot(h, wnt_ref[...]) + _dot(aglo_ref[...], wnblo_ref[...])
       + _dot(aghi_ref[...], wnbhi_ref[...]) + bn_ref[...])
  h1 = h + jnp.maximum(u, 0.0)
  h1_ref[...] = h1
  a = _dot(h1, wet_ref[...])
  b = _dot(h1, web_ref[...]) + be_ref[...]
  alo_ref[...] = a[:, :H]
  ahi_ref[...] = a[:, H:]
  blo_ref[...] = b[:, :H]
  bhi_ref[...] = b[:, H:]


def _final_body(h_ref, aglo_ref, aghi_ref, wnt_ref, wnblo_ref, wnbhi_ref,
                bn_ref, seg_ref, w1_ref, b1_ref, w2_ref, b2_ref,
                w3_ref, b3_ref, out_ref, gsum_ref, cnt_ref):
  h = h_ref[...]
  u = (_dot(h, wnt_ref[...]) + _dot(aglo_ref[...], wnblo_ref[...])
       + _dot(aghi_ref[...], wnbhi_ref[...]) + bn_ref[...])
  h2 = h + jnp.maximum(u, 0.0)
  seg = seg_ref[0, 0, :]
  gids = lax.broadcasted_iota(jnp.int32, (G, BR), 0)
  m = (seg[None, :] == gids).astype(jnp.float32)

  @pl.when(pl.program_id(0) == 0)
  def _():
    gsum_ref[...] = jnp.zeros_like(gsum_ref)
    cnt_ref[...] = jnp.zeros_like(cnt_ref)

  gsum_ref[...] += _dot(m, h2)
  cnt_ref[...] += jnp.broadcast_to(jnp.sum(m, axis=1, keepdims=True), (G, D))

  @pl.when(pl.program_id(0) == NB - 1)
  def _():
    g = gsum_ref[...] / jnp.maximum(cnt_ref[...], 1.0)
    x = jnp.maximum(_dot(g, w1_ref[...]) + b1_ref[...], 0.0)
    x = jnp.maximum(_dot(x, w2_ref[...]) + b2_ref[...], 0.0)
    out_ref[...] = _dot(x, w3_ref[...]) + b3_ref[...]


def _row_spec(width):
  return pl.BlockSpec((BR, width), lambda i: (i, 0))


def _full_spec(shape):
  return pl.BlockSpec(shape, lambda i: tuple(0 for _ in shape))


_prep_call = pl.pallas_call(
    _prep_body,
    grid=(NB,),
    in_specs=[_row_spec(D), _row_spec(D), _full_spec((D, D)),
              _full_spec((D, D)), _full_spec((1, D))],
    out_specs=[_row_spec(D), _row_spec(H), _row_spec(H),
               _row_spec(H), _row_spec(H)],
    out_shape=[jax.ShapeDtypeStruct((N, D), jnp.float32)]
    + [jax.ShapeDtypeStruct((N, H), jnp.float32)] * 4,
)

_update_call = pl.pallas_call(
    _update_body,
    grid=(NB,),
    in_specs=[_row_spec(D), _row_spec(H), _row_spec(H),
              _full_spec((D, D)), _full_spec((H, D)), _full_spec((H, D)),
              _full_spec((1, D)), _full_spec((D, D)), _full_spec((D, D)),
              _full_spec((1, D))],
    out_specs=[_row_spec(D), _row_spec(H), _row_spec(H),
               _row_spec(H), _row_spec(H)],
    out_shape=[jax.ShapeDtypeStruct((N, D), jnp.float32)]
    + [jax.ShapeDtypeStruct((N, H), jnp.float32)] * 4,
)

_final_call = pl.pallas_call(
    _final_body,
    grid=(NB,),
    in_specs=[_row_spec(D), _row_spec(H), _row_spec(H),
              _full_spec((D, D)), _full_spec((H, D)), _full_spec((H, D)),
              _full_spec((1, D)), pl.BlockSpec((1, 1, BR), lambda i: (i, 0, 0)),
              _full_spec((D, D)), _full_spec((1, D)), _full_spec((D, D)),
              _full_spec((1, D)), _full_spec((D, H)), _full_spec((1, H))],
    out_specs=_full_spec((G, H)),
    out_shape=jax.ShapeDtypeStruct((G, H), jnp.float32),
    scratch_shapes=[pltpu.VMEM((G, D), jnp.float32),
                    pltpu.VMEM((G, D), jnp.float32)],
)


@jax.jit
def _run(state_batch, action_batch, edge_index, segment_ids,
         We0, be0, Wn0, bn0, We1, be1, Wn1, bn1, W1, b1, W2, b2, W3, b3):
  src = edge_index[0]
  dst = edge_index[1]
  zpad = jnp.zeros((EP - E,), jnp.int32)
  src_g = jnp.concatenate([src, zpad]).reshape(NS * NCH, K)
  dst_g = jnp.concatenate([dst, zpad]).reshape(NS * NCH, K)
  dst_s = jnp.concatenate([dst, jnp.full((EP - E,), N, jnp.int32)]
                          ).reshape(NS * NCH, K)
  idx_all = jnp.stack([src_g, dst_g, dst_s], axis=1)
  seg3 = segment_ids.reshape(NB, 1, BR)

  h0, a0lo, a0hi, b0lo, b0hi = _prep_call(
      state_batch, action_batch, We0[:D], We0[D:], be0.reshape(1, D))
  ag0lo, ag0hi = _edge_call(a0lo, a0hi, b0lo, b0hi, idx_all)
  h1, a1lo, a1hi, b1lo, b1hi = _update_call(
      h0, ag0lo, ag0hi, Wn0[:D], Wn0[D:D + H], Wn0[D + H:], bn0.reshape(1, D),
      We1[:D], We1[D:], be1.reshape(1, D))
  ag1lo, ag1hi = _edge_call(a1lo, a1hi, b1lo, b1hi, idx_all)
  w3p = jnp.pad(W3, ((0, 0), (0, H - 1)))
  b3p = jnp.pad(b3.reshape(1, 1), ((0, 0), (0, H - 1)))
  out = _final_call(
      h1, ag1lo, ag1hi, Wn1[:D], Wn1[D:D + H], Wn1[D + H:], bn1.reshape(1, D),
      seg3, W1, b1.reshape(1, D), W2, b2.reshape(1, D), w3p, b3p)
  return out[:, :1]


def kernel(state_batch, action_batch, edge_index, segment_ids,
           We0, be0, Wn0, bn0, We1, be1, Wn1, bn1, W1, b1, W2, b2, W3, b3):
  return _run(state_batch, action_batch, edge_index, segment_ids,
              We0, be0, Wn0, bn0, We1, be1, Wn1, bn1, W1, b1, W2, b2, W3, b3)
